# baseline (speedup 1.0000x reference)
import jax
import jax.numpy as jnp
from jax.experimental import pallas as pl
from jax.experimental.pallas import tpu as pltpu

NUM_CLASSES = 401
TOP_K = 16
THRESH = 0.05
BLOCK_ROWS = 400


_CBITS = 0x3D4CCCCD


def _topk_body(x_ref, feat_ref, vals_ref, idx_ref, feat_out_ref, sem):
    i = pl.program_id(0)
    cp = pltpu.make_async_copy(
        feat_ref.at[pl.ds(i * BLOCK_ROWS, BLOCK_ROWS)],
        feat_out_ref.at[pl.ds(i * BLOCK_ROWS, BLOCK_ROWS)],
        sem,
    )
    cp.start()
    x = x_ref[...]
    m = jnp.max(x, axis=-1, keepdims=True)
    e = jnp.exp(x - m)
    s = jnp.sum(e, axis=-1, keepdims=True)
    r = 1.0 / s
    p = e * r
    col = jax.lax.broadcasted_iota(jnp.int32, p.shape, 1)
    mask = (col != 0) & (p > THRESH)
    bits = jax.lax.bitcast_convert_type(e, jnp.int32)
    valpart = ((bits - _CBITS) >> 4) + 1
    key = jnp.where(mask, valpart << 9, 0) | (511 - col)
    keys = []
    for _ in range(TOP_K):
        mx = jnp.max(key, axis=-1, keepdims=True)
        keys.append(mx)
        key = jnp.where(key == mx, -1, key)
    k16 = jnp.concatenate(keys, axis=-1)
    vp = k16 >> 9
    e_rec = jax.lax.bitcast_convert_type(
        ((vp - 1) << 4) + (_CBITS + 8), jnp.float32)
    vals_ref[...] = jnp.where(vp > 0, e_rec * r, 0.0)
    idx_ref[...] = 511 - (k16 & 511)
    cp.wait()


def _run_topk(x, features, interpret=False):
    rows = x.shape[0]
    grid = (rows // BLOCK_ROWS,)
    return pl.pallas_call(
        _topk_body,
        grid=grid,
        in_specs=[
            pl.BlockSpec((BLOCK_ROWS, NUM_CLASSES), lambda i: (i, 0)),
            pl.BlockSpec(memory_space=pl.ANY),
        ],
        out_specs=[
            pl.BlockSpec((BLOCK_ROWS, TOP_K), lambda i: (i, 0)),
            pl.BlockSpec((BLOCK_ROWS, TOP_K), lambda i: (i, 0)),
            pl.BlockSpec(memory_space=pl.ANY),
        ],
        out_shape=[
            jax.ShapeDtypeStruct((rows, TOP_K), jnp.float32),
            jax.ShapeDtypeStruct((rows, TOP_K), jnp.int32),
            jax.ShapeDtypeStruct(features.shape, features.dtype),
        ],
        scratch_shapes=[pltpu.SemaphoreType.DMA],
        compiler_params=pltpu.CompilerParams(
            dimension_semantics=("arbitrary",)),
        interpret=interpret,
    )(x, features)


def kernel(x, features):
    attr_scores, attr_labels, feat_out = _run_topk(x, features)
    return attr_scores, attr_labels, feat_out


# speedup vs baseline: 1.0109x; 1.0109x over previous
import jax
import jax.numpy as jnp
from jax.experimental import pallas as pl
from jax.experimental.pallas import tpu as pltpu

NUM_CLASSES = 401
TOP_K = 16
THRESH = 0.05
BLOCK_ROWS = 400


_CBITS = 0x3D4CCCCD


def _topk_body(x_ref, feat_ref, vals_ref, idx_ref, feat_out_ref, sem):
    i = pl.program_id(0)

    @pl.when(i == 0)
    def _():
        pltpu.make_async_copy(feat_ref, feat_out_ref, sem).start()

    x = x_ref[...]
    m = jnp.max(x, axis=-1, keepdims=True)
    e = jnp.exp(x - m)
    s = jnp.sum(e, axis=-1, keepdims=True)
    r = 1.0 / s
    p = e * r
    col = jax.lax.broadcasted_iota(jnp.int32, p.shape, 1)
    mask = (col != 0) & (p > THRESH)
    bits = jax.lax.bitcast_convert_type(e, jnp.int32)
    valpart = ((bits - _CBITS) >> 4) + 1
    key = jnp.where(mask, valpart << 9, 0) | (511 - col)
    keys = []
    for _ in range(TOP_K):
        mx = jnp.max(key, axis=-1, keepdims=True)
        keys.append(mx)
        key = jnp.where(key == mx, -1, key)
    k16 = jnp.concatenate(keys, axis=-1)
    vp = k16 >> 9
    e_rec = jax.lax.bitcast_convert_type(
        ((vp - 1) << 4) + (_CBITS + 8), jnp.float32)
    vals_ref[...] = jnp.where(vp > 0, e_rec * r, 0.0)
    idx_ref[...] = 511 - (k16 & 511)

    @pl.when(i == pl.num_programs(0) - 1)
    def _():
        pltpu.make_async_copy(feat_ref, feat_out_ref, sem).wait()


def _run_topk(x, features, interpret=False):
    rows = x.shape[0]
    grid = (rows // BLOCK_ROWS,)
    return pl.pallas_call(
        _topk_body,
        grid=grid,
        in_specs=[
            pl.BlockSpec((BLOCK_ROWS, NUM_CLASSES), lambda i: (i, 0)),
            pl.BlockSpec(memory_space=pl.ANY),
        ],
        out_specs=[
            pl.BlockSpec((BLOCK_ROWS, TOP_K), lambda i: (i, 0)),
            pl.BlockSpec((BLOCK_ROWS, TOP_K), lambda i: (i, 0)),
            pl.BlockSpec(memory_space=pl.ANY),
        ],
        out_shape=[
            jax.ShapeDtypeStruct((rows, TOP_K), jnp.float32),
            jax.ShapeDtypeStruct((rows, TOP_K), jnp.int32),
            jax.ShapeDtypeStruct(features.shape, features.dtype),
        ],
        scratch_shapes=[pltpu.SemaphoreType.DMA],
        compiler_params=pltpu.CompilerParams(
            dimension_semantics=("arbitrary",)),
        interpret=interpret,
    )(x, features)


def kernel(x, features):
    attr_scores, attr_labels, feat_out = _run_topk(x, features)
    return attr_scores, attr_labels, feat_out


# speedup vs baseline: 21.1390x; 20.9117x over previous
import jax
import jax.numpy as jnp
from jax.experimental import pallas as pl
from jax.experimental.pallas import tpu as pltpu

NUM_CLASSES = 401
TOP_K = 16
THRESH = 0.05
BLOCK_ROWS = 400


_CBITS = 0x3D4CCCCD


def _topk_body(x_ref, feat_ref, vals_ref, idx_ref, feat_out_ref):
    feat_out_ref[...] = feat_ref[...]
    x = x_ref[...]
    m = jnp.max(x, axis=-1, keepdims=True)
    e = jnp.exp(x - m)
    s = jnp.sum(e, axis=-1, keepdims=True)
    r = 1.0 / s
    p = e * r
    col = jax.lax.broadcasted_iota(jnp.int32, p.shape, 1)
    mask = (col != 0) & (p > THRESH)
    bits = jax.lax.bitcast_convert_type(e, jnp.int32)
    valpart = ((bits - _CBITS) >> 4) + 1
    key = jnp.where(mask, valpart << 9, 0) | (511 - col)
    keys = []
    for _ in range(TOP_K):
        mx = jnp.max(key, axis=-1, keepdims=True)
        keys.append(mx)
        key = jnp.where(key == mx, -1, key)
    k16 = jnp.concatenate(keys, axis=-1)
    vp = k16 >> 9
    e_rec = jax.lax.bitcast_convert_type(
        ((vp - 1) << 4) + (_CBITS + 8), jnp.float32)
    vals_ref[...] = jnp.where(vp > 0, e_rec * r, 0.0)
    idx_ref[...] = 511 - (k16 & 511)


def _run_topk(x, features, interpret=False):
    rows = x.shape[0]
    grid = (rows // BLOCK_ROWS,)
    return pl.pallas_call(
        _topk_body,
        grid=grid,
        in_specs=[
            pl.BlockSpec((BLOCK_ROWS, NUM_CLASSES), lambda i: (i, 0)),
            pl.BlockSpec((BLOCK_ROWS, 2048), lambda i: (i, 0)),
        ],
        out_specs=[
            pl.BlockSpec((BLOCK_ROWS, TOP_K), lambda i: (i, 0)),
            pl.BlockSpec((BLOCK_ROWS, TOP_K), lambda i: (i, 0)),
            pl.BlockSpec((BLOCK_ROWS, 2048), lambda i: (i, 0)),
        ],
        out_shape=[
            jax.ShapeDtypeStruct((rows, TOP_K), jnp.float32),
            jax.ShapeDtypeStruct((rows, TOP_K), jnp.int32),
            jax.ShapeDtypeStruct(features.shape, features.dtype),
        ],
        compiler_params=pltpu.CompilerParams(
            dimension_semantics=("arbitrary",)),
        interpret=interpret,
    )(x, features)


def kernel(x, features):
    attr_scores, attr_labels, feat_out = _run_topk(x, features)
    return attr_scores, attr_labels, feat_out


# speedup vs baseline: 22.8810x; 1.0824x over previous
import jax
import jax.numpy as jnp
from jax.experimental import pallas as pl
from jax.experimental.pallas import tpu as pltpu

NUM_CLASSES = 401
TOP_K = 16
THRESH = 0.05
BLOCK_ROWS = 800


_CBITS = 0x3D4CCCCD


def _topk_body(x_ref, feat_ref, vals_ref, idx_ref, feat_out_ref):
    feat_out_ref[...] = feat_ref[...]
    x = x_ref[...]
    m = jnp.max(x, axis=-1, keepdims=True)
    e = jnp.exp(x - m)
    s = jnp.sum(e, axis=-1, keepdims=True)
    r = 1.0 / s
    p = e * r
    col = jax.lax.broadcasted_iota(jnp.int32, p.shape, 1)
    mask = (col != 0) & (p > THRESH)
    bits = jax.lax.bitcast_convert_type(e, jnp.int32)
    valpart = ((bits - _CBITS) >> 4) + 1
    key = jnp.where(mask, valpart << 9, 0) | (511 - col)
    keys = []
    for _ in range(TOP_K):
        mx = jnp.max(key, axis=-1, keepdims=True)
        keys.append(mx)
        key = jnp.where(key == mx, -1, key)
    k16 = jnp.concatenate(keys, axis=-1)
    vp = k16 >> 9
    e_rec = jax.lax.bitcast_convert_type(
        ((vp - 1) << 4) + (_CBITS + 8), jnp.float32)
    vals_ref[...] = jnp.where(vp > 0, e_rec * r, 0.0)
    idx_ref[...] = 511 - (k16 & 511)


def _run_topk(x, features, interpret=False):
    rows = x.shape[0]
    grid = (rows // BLOCK_ROWS,)
    return pl.pallas_call(
        _topk_body,
        grid=grid,
        in_specs=[
            pl.BlockSpec((BLOCK_ROWS, NUM_CLASSES), lambda i: (i, 0)),
            pl.BlockSpec((BLOCK_ROWS, 2048), lambda i: (i, 0)),
        ],
        out_specs=[
            pl.BlockSpec((BLOCK_ROWS, TOP_K), lambda i: (i, 0)),
            pl.BlockSpec((BLOCK_ROWS, TOP_K), lambda i: (i, 0)),
            pl.BlockSpec((BLOCK_ROWS, 2048), lambda i: (i, 0)),
        ],
        out_shape=[
            jax.ShapeDtypeStruct((rows, TOP_K), jnp.float32),
            jax.ShapeDtypeStruct((rows, TOP_K), jnp.int32),
            jax.ShapeDtypeStruct(features.shape, features.dtype),
        ],
        compiler_params=pltpu.CompilerParams(
            dimension_semantics=("arbitrary",)),
        interpret=interpret,
    )(x, features)


def kernel(x, features):
    attr_scores, attr_labels, feat_out = _run_topk(x, features)
    return attr_scores, attr_labels, feat_out
